# Initial kernel scaffold; baseline (speedup 1.0000x reference)
import jax
import jax.numpy as jnp
from jax.experimental import pallas as pl


def kernel(z, edge_index):
    raise NotImplementedError("write your pallas kernel here")


# speedup vs baseline: 1.0814x; 1.0814x over previous
import jax
import jax.numpy as jnp
from jax import lax
from jax.experimental import pallas as pl
from jax.experimental.pallas import tpu as pltpu
from jax.experimental.pallas import tpu_sc as plsc

N_NODES = 10000
D = 256
N_EDGES = 160000
NC = 2
NS = 16
NW = NC * NS
EPW = 5120
PAD_E = EPW * NW
C = 64
NCHUNK = EPW // C
LG = 16
NSEG = D // LG
NGRP = C // LG


def _permute(x, idx):
    dnums = lax.GatherDimensionNumbers(
        offset_dims=(), collapsed_slice_dims=(0,), start_index_map=(0,))
    return lax.gather(x, idx[:, None], dnums, (1,),
                      mode=lax.GatherScatterMode.PROMISE_IN_BOUNDS)


def _body(z_hbm, src_hbm, dst_hbm, out_hbm,
          sidx, didx, srow, drow, outv, sem_s, sem_d):
    wid = lax.axis_index("s") * NC + lax.axis_index("c")
    base = wid * EPW

    pltpu.sync_copy(src_hbm.at[pl.ds(base, EPW)], sidx)
    pltpu.sync_copy(dst_hbm.at[pl.ds(base, EPW)], didx)

    lane = lax.iota(jnp.int32, LG)
    perms = [(lane + s) & (LG - 1) for s in (8, 4, 2, 1)]

    def chunk_body(g, carry):
        cs = pltpu.async_copy(z_hbm.at[sidx.at[pl.ds(g * C, C)]], srow, sem_s)
        cd = pltpu.async_copy(z_hbm.at[didx.at[pl.ds(g * C, C)]], drow, sem_d)
        cs.wait()
        cd.wait()

        def group_body(q, carry2):
            gvec = jnp.zeros((LG,), jnp.float32)
            for i in range(LG):
                e = q * LG + i
                acc = srow[e, pl.ds(0, LG)] * drow[e, pl.ds(0, LG)]
                for j in range(1, NSEG):
                    acc = acc + srow[e, pl.ds(j * LG, LG)] * drow[e, pl.ds(j * LG, LG)]
                for p in perms:
                    acc = acc + _permute(acc, p)
                gvec = jnp.where(lane == i, acc, gvec)
            outv[pl.ds(g * C + q * LG, LG)] = 1.0 / (1.0 + jnp.exp(-gvec))
            return carry2

        lax.fori_loop(0, NGRP, group_body, 0)
        return carry

    lax.fori_loop(0, NCHUNK, chunk_body, 0)
    pltpu.sync_copy(outv, out_hbm.at[pl.ds(base, EPW)])


@jax.jit
def _decode(z, src, dst):
    mesh = plsc.VectorSubcoreMesh(core_axis_name="c", subcore_axis_name="s")
    f = pl.kernel(
        _body,
        mesh=mesh,
        out_type=jax.ShapeDtypeStruct((PAD_E,), jnp.float32),
        scratch_types=[
            pltpu.VMEM((EPW,), jnp.int32),
            pltpu.VMEM((EPW,), jnp.int32),
            pltpu.VMEM((C, D), jnp.float32),
            pltpu.VMEM((C, D), jnp.float32),
            pltpu.VMEM((EPW,), jnp.float32),
            pltpu.SemaphoreType.DMA,
            pltpu.SemaphoreType.DMA,
        ],
    )
    return f(z, src, dst)


def kernel(z, edge_index):
    src = jnp.pad(edge_index[0].astype(jnp.int32), (0, PAD_E - N_EDGES))
    dst = jnp.pad(edge_index[1].astype(jnp.int32), (0, PAD_E - N_EDGES))
    return _decode(z, src, dst)[:N_EDGES]


# speedup vs baseline: 1.5868x; 1.4674x over previous
import jax
import jax.numpy as jnp
from jax import lax
from jax.experimental import pallas as pl
from jax.experimental.pallas import tpu as pltpu
from jax.experimental.pallas import tpu_sc as plsc

N_NODES = 10000
D = 256
N_EDGES = 160000
NC = 2
NS = 16
NW = NC * NS
EPW = 5120
PAD_E = EPW * NW
C = 64
NCHUNK = EPW // C
LG = 16
NSEG = D // LG
NGRP = C // LG


def _permute(x, idx):
    dnums = lax.GatherDimensionNumbers(
        offset_dims=(), collapsed_slice_dims=(0,), start_index_map=(0,))
    return lax.gather(x, idx[:, None], dnums, (1,),
                      mode=lax.GatherScatterMode.PROMISE_IN_BOUNDS)


def _body(z_hbm, src_hbm, dst_hbm, out_hbm,
          sidx, didx, srow, drow, outv, sem_s0, sem_s1, sem_d0, sem_d1):
    wid = lax.axis_index("s") * NC + lax.axis_index("c")
    base = wid * EPW

    pltpu.sync_copy(src_hbm.at[pl.ds(base, EPW)], sidx)
    pltpu.sync_copy(dst_hbm.at[pl.ds(base, EPW)], didx)

    lane = lax.iota(jnp.int32, LG)
    perms = [(lane + s) & (LG - 1) for s in (8, 4, 2, 1)]
    sems = ((sem_s0, sem_d0), (sem_s1, sem_d1))

    def start(g, b):
        pltpu.async_copy(z_hbm.at[sidx.at[pl.ds(g * C, C)]], srow.at[b],
                         sems[b][0])
        pltpu.async_copy(z_hbm.at[didx.at[pl.ds(g * C, C)]], drow.at[b],
                         sems[b][1])

    def wait(g, b):
        pltpu.make_async_copy(z_hbm.at[sidx.at[pl.ds(g * C, C)]], srow.at[b],
                              sems[b][0]).wait()
        pltpu.make_async_copy(z_hbm.at[didx.at[pl.ds(g * C, C)]], drow.at[b],
                              sems[b][1]).wait()

    def compute(g, b):
        def group_body(q, carry2):
            gvec = jnp.zeros((LG,), jnp.float32)
            for i in range(LG):
                e = q * LG + i
                accs = [srow[b, e, pl.ds(a * LG, LG)] * drow[b, e, pl.ds(a * LG, LG)]
                        for a in range(4)]
                for j in range(4, NSEG):
                    a = j & 3
                    accs[a] = accs[a] + (srow[b, e, pl.ds(j * LG, LG)]
                                         * drow[b, e, pl.ds(j * LG, LG)])
                acc = (accs[0] + accs[1]) + (accs[2] + accs[3])
                for p in perms:
                    acc = acc + _permute(acc, p)
                gvec = jnp.where(lane == i, acc, gvec)
            outv[pl.ds(g * C + q * LG, LG)] = 1.0 / (1.0 + jnp.exp(-gvec))
            return carry2

        lax.fori_loop(0, NGRP, group_body, 0)

    start(0, 0)
    start(1, 1)

    def outer(t, carry):
        for b in range(2):
            g = 2 * t + b
            wait(g, b)
            compute(g, b)

            @pl.when(g + 2 < NCHUNK)
            def _():
                start(g + 2, b)
        return carry

    lax.fori_loop(0, NCHUNK // 2, outer, 0)
    pltpu.sync_copy(outv, out_hbm.at[pl.ds(base, EPW)])


@jax.jit
def _decode(z, src, dst):
    mesh = plsc.VectorSubcoreMesh(core_axis_name="c", subcore_axis_name="s")
    f = pl.kernel(
        _body,
        mesh=mesh,
        out_type=jax.ShapeDtypeStruct((PAD_E,), jnp.float32),
        scratch_types=[
            pltpu.VMEM((EPW,), jnp.int32),
            pltpu.VMEM((EPW,), jnp.int32),
            pltpu.VMEM((2, C, D), jnp.float32),
            pltpu.VMEM((2, C, D), jnp.float32),
            pltpu.VMEM((EPW,), jnp.float32),
            pltpu.SemaphoreType.DMA,
            pltpu.SemaphoreType.DMA,
            pltpu.SemaphoreType.DMA,
            pltpu.SemaphoreType.DMA,
        ],
    )
    return f(z, src, dst)


def kernel(z, edge_index):
    src = jnp.pad(edge_index[0].astype(jnp.int32), (0, PAD_E - N_EDGES))
    dst = jnp.pad(edge_index[1].astype(jnp.int32), (0, PAD_E - N_EDGES))
    return _decode(z, src, dst)[:N_EDGES]
